# baseline (speedup 1.0000x reference)
import jax
import jax.numpy as jnp
from jax.experimental import pallas as pl
from jax.experimental.pallas import tpu as pltpu

_N = 2048
_CE = 64
_BLK = 512
_EPS = 1e-05


def _corr_body(qt_ref, ke_ref, v_ref, corr_ref):
    qt = qt_ref[0]
    k = ke_ref[0]
    v = v_ref[0]
    inner = -2.0 * jax.lax.dot_general(
        qt, k, (((1,), (0,)), ((), ())),
        preferred_element_type=jnp.float32,
        precision=jax.lax.Precision.DEFAULT)
    xx = jnp.sum(qt * qt, axis=1, keepdims=True)
    yy = jnp.sum(k * k, axis=0, keepdims=True)
    logits = -xx - inner - yy
    m = jnp.max(logits, axis=1, keepdims=True)
    p = jnp.exp(logits - m)
    s = jnp.sum(p, axis=1, keepdims=True)
    scores = p / s
    corr_ref[0] = jax.lax.dot_general(
        v, scores, (((1,), (1,)), ((), ())),
        preferred_element_type=jnp.float32,
        precision=jax.lax.Precision.DEFAULT)


def _cov_body(a_ref, b_ref, cov_ref, ca_ref, cb_ref):
    n = a_ref.shape[-1]
    w = (1.0 / n) / (1.0 + _EPS)
    a = a_ref[0]
    b = b_ref[0]
    ca = jnp.sum(a, axis=1, keepdims=True) * w
    cb = jnp.sum(b, axis=1, keepdims=True) * w
    ac = a - ca
    bc = b - cb
    cov = jax.lax.dot_general(
        ac, bc, (((1,), (1,)), ((), ())),
        preferred_element_type=jnp.float32,
        precision=jax.lax.Precision.DEFAULT) * w
    cov_ref[0] = cov
    ca_ref[0, 0] = ca[:, 0]
    cb_ref[0, 0] = cb[:, 0]


def kernel(src, tgt, src_embedding, tgt_embedding, positive_T):
    batch, posi_num, num_points, cdim = tgt.shape
    bp = batch * posi_num
    cemb = tgt_embedding.shape[2]

    src_ = jnp.swapaxes(src, -2, -1).reshape(bp, cdim, num_points)
    tgt_ = jnp.swapaxes(tgt, -2, -1).reshape(bp, cdim, num_points)
    src_emb_t = jnp.broadcast_to(
        jnp.swapaxes(jnp.squeeze(src_embedding, -1), -2, -1),
        (batch, posi_num, num_points, cemb)).reshape(bp, num_points, cemb)
    tgt_emb = jnp.squeeze(tgt_embedding, -1).reshape(bp, cemb, num_points)

    nblk = num_points // _BLK
    corr = pl.pallas_call(
        _corr_body,
        grid=(bp, nblk),
        in_specs=[
            pl.BlockSpec((1, _BLK, cemb), lambda b, j: (b, j, 0)),
            pl.BlockSpec((1, cemb, num_points), lambda b, j: (b, 0, 0)),
            pl.BlockSpec((1, cdim, num_points), lambda b, j: (b, 0, 0)),
        ],
        out_specs=pl.BlockSpec((1, cdim, _BLK), lambda b, j: (b, 0, j)),
        out_shape=jax.ShapeDtypeStruct((bp, cdim, num_points), jnp.float32),
        compiler_params=pltpu.CompilerParams(
            dimension_semantics=("parallel", "parallel")),
    )(src_emb_t, tgt_emb, tgt_)

    cov, ca, cb = pl.pallas_call(
        _cov_body,
        grid=(bp,),
        in_specs=[
            pl.BlockSpec((1, cdim, num_points), lambda b: (b, 0, 0)),
            pl.BlockSpec((1, cdim, num_points), lambda b: (b, 0, 0)),
        ],
        out_specs=[
            pl.BlockSpec((1, cdim, cdim), lambda b: (b, 0, 0)),
            pl.BlockSpec((1, 1, cdim), lambda b: (b, 0, 0)),
            pl.BlockSpec((1, 1, cdim), lambda b: (b, 0, 0)),
        ],
        out_shape=[
            jax.ShapeDtypeStruct((bp, cdim, cdim), jnp.float32),
            jax.ShapeDtypeStruct((bp, 1, cdim), jnp.float32),
            jax.ShapeDtypeStruct((bp, 1, cdim), jnp.float32),
        ],
        compiler_params=pltpu.CompilerParams(
            dimension_semantics=("parallel",)),
    )(src_, corr)

    u, _, vh = jnp.linalg.svd(cov, full_matrices=True)
    v = jnp.swapaxes(vh, -1, -2)
    rot_pos = v @ jnp.swapaxes(u, -1, -2)
    v_neg = v.at[:, :, 2].multiply(-1.0)
    rot_neg = v_neg @ jnp.swapaxes(u, -1, -2)
    det = jnp.linalg.det(rot_pos)
    rot_mat = jnp.where(det[:, None, None] > 0, rot_pos, rot_neg)
    ca = ca.reshape(bp, cdim)
    cb = cb.reshape(bp, cdim)
    translation = (-rot_mat @ ca[:, :, None] + cb[:, :, None]).reshape(bp, 3)
    return (rot_mat, translation, src_, corr)


# speedup vs baseline: 1.2326x; 1.2326x over previous
import jax
import jax.numpy as jnp
from jax.experimental import pallas as pl
from jax.experimental.pallas import tpu as pltpu

_BLK = 512
_EPS = 1e-05


def _corr_body(qt_ref, ke_ref, v_ref, corr_ref):
    qt = qt_ref[0]
    k = ke_ref[0]
    v = v_ref[0]
    inner = -2.0 * jax.lax.dot_general(
        qt, k, (((1,), (0,)), ((), ())),
        preferred_element_type=jnp.float32)
    xx = jnp.sum(qt * qt, axis=1, keepdims=True)
    yy = jnp.sum(k * k, axis=0, keepdims=True)
    logits = -xx - inner - yy
    m = jnp.max(logits, axis=1, keepdims=True)
    p = jnp.exp(logits - m)
    s = jnp.sum(p, axis=1, keepdims=True)
    scores = p / s
    corr_ref[0] = jax.lax.dot_general(
        v, scores, (((1,), (1,)), ((), ())),
        preferred_element_type=jnp.float32)


def _g(mat, i, j):
    return mat[i:i + 1, j:j + 1]


def _col(mat, j):
    return mat[:, j:j + 1]


def _e_row(j, dtype):
    lane = jax.lax.broadcasted_iota(jnp.int32, (1, 3), 1)
    return jnp.where(lane == j, 1.0, 0.0).astype(dtype)


def _jacobi_rot(A, V, p, q):
    ep = _e_row(p, A.dtype)
    eq = _e_row(q, A.dtype)
    ap = _col(A, p)
    aq = _col(A, q)
    vp = _col(V, p)
    vq = _col(V, q)
    alpha = jnp.sum(ap * ap, axis=0, keepdims=True)
    beta = jnp.sum(aq * aq, axis=0, keepdims=True)
    gamma = jnp.sum(ap * aq, axis=0, keepdims=True)
    absg = jnp.abs(gamma)
    safe_g = jnp.where(absg > 0, gamma, 1.0)
    tau = (beta - alpha) / (2.0 * safe_g)
    sign = jnp.where(tau >= 0, 1.0, -1.0)
    t = sign / (jnp.abs(tau) + jnp.sqrt(1.0 + tau * tau))
    t = jnp.where(absg > 0, t, 0.0)
    c = jax.lax.rsqrt(1.0 + t * t)
    s = t * c
    ap2 = c * ap - s * aq
    aq2 = s * ap + c * aq
    vp2 = c * vp - s * vq
    vq2 = s * vp + c * vq
    A2 = A + (ap2 - ap) * ep + (aq2 - aq) * eq
    V2 = V + (vp2 - vp) * ep + (vq2 - vq) * eq
    return A2, V2


def _pose_body(a_ref, b_ref, rot_ref, tr_ref):
    n = a_ref.shape[-1]
    w = (1.0 / n) / (1.0 + _EPS)
    a = a_ref[0]
    b = b_ref[0]
    ca = jnp.sum(a, axis=1, keepdims=True) * w
    cb = jnp.sum(b, axis=1, keepdims=True) * w
    ac = a - ca
    bc = b - cb
    cov = jax.lax.dot_general(
        ac, bc, (((1,), (1,)), ((), ())),
        preferred_element_type=jnp.float32) * w

    A = cov
    row_i = jax.lax.broadcasted_iota(jnp.int32, (3, 3), 0)
    col_i = jax.lax.broadcasted_iota(jnp.int32, (3, 3), 1)
    V = jnp.where(row_i == col_i, 1.0, 0.0).astype(cov.dtype)
    for _ in range(6):
        for (p, q) in ((0, 1), (0, 2), (1, 2)):
            A, V = _jacobi_rot(A, V, p, q)
    s2 = jnp.sum(A * A, axis=0, keepdims=True)
    s = jnp.sqrt(s2)
    U = A / s
    rot_pos = jax.lax.dot_general(V, U, (((1,), (1,)), ((), ())))
    s0, s1, s2v = _g(s2, 0, 0), _g(s2, 0, 1), _g(s2, 0, 2)
    m0 = jnp.where(jnp.logical_and(s0 <= s1, s0 <= s2v), 1.0, 0.0)
    m1 = jnp.where(jnp.logical_and(s1 < s0, s1 <= s2v), 1.0, 0.0)
    m2 = jnp.where(jnp.logical_and(s2v < s0, s2v < s1), 1.0, 0.0)
    sel = (m0 * _e_row(0, cov.dtype)
           + m1 * _e_row(1, cov.dtype)
           + m2 * _e_row(2, cov.dtype))
    v3 = jax.lax.dot_general(V, sel, (((1,), (1,)), ((), ())))
    u3t = jax.lax.dot_general(sel, U, (((1,), (1,)), ((), ())))
    rot_neg = rot_pos - 2.0 * (v3 * u3t)
    det = (_g(cov, 0, 0) * (_g(cov, 1, 1) * _g(cov, 2, 2) - _g(cov, 1, 2) * _g(cov, 2, 1))
           - _g(cov, 0, 1) * (_g(cov, 1, 0) * _g(cov, 2, 2) - _g(cov, 1, 2) * _g(cov, 2, 0))
           + _g(cov, 0, 2) * (_g(cov, 1, 0) * _g(cov, 2, 1) - _g(cov, 1, 1) * _g(cov, 2, 0)))
    pos_w = jnp.where(det > 0, 1.0, 0.0)
    rot = rot_neg + pos_w * (rot_pos - rot_neg)
    trans = cb - jax.lax.dot_general(rot, ca, (((1,), (0,)), ((), ())))
    rot_ref[0] = rot
    tr_ref[0] = trans


def kernel(src, tgt, src_embedding, tgt_embedding, positive_T):
    batch, posi_num, num_points, cdim = tgt.shape
    bp = batch * posi_num
    cemb = tgt_embedding.shape[2]

    src_ = jnp.swapaxes(src, -2, -1).reshape(bp, cdim, num_points)
    tgt_ = jnp.swapaxes(tgt, -2, -1).reshape(bp, cdim, num_points)
    src_emb_t = jnp.broadcast_to(
        jnp.swapaxes(jnp.squeeze(src_embedding, -1), -2, -1),
        (batch, posi_num, num_points, cemb)).reshape(bp, num_points, cemb)
    tgt_emb = jnp.squeeze(tgt_embedding, -1).reshape(bp, cemb, num_points)

    nblk = num_points // _BLK
    corr = pl.pallas_call(
        _corr_body,
        grid=(bp, nblk),
        in_specs=[
            pl.BlockSpec((1, _BLK, cemb), lambda b, j: (b, j, 0)),
            pl.BlockSpec((1, cemb, num_points), lambda b, j: (b, 0, 0)),
            pl.BlockSpec((1, cdim, num_points), lambda b, j: (b, 0, 0)),
        ],
        out_specs=pl.BlockSpec((1, cdim, _BLK), lambda b, j: (b, 0, j)),
        out_shape=jax.ShapeDtypeStruct((bp, cdim, num_points), jnp.float32),
        compiler_params=pltpu.CompilerParams(
            dimension_semantics=("parallel", "parallel")),
    )(src_emb_t, tgt_emb, tgt_)

    rot_mat, trans = pl.pallas_call(
        _pose_body,
        grid=(bp,),
        in_specs=[
            pl.BlockSpec((1, cdim, num_points), lambda b: (b, 0, 0)),
            pl.BlockSpec((1, cdim, num_points), lambda b: (b, 0, 0)),
        ],
        out_specs=[
            pl.BlockSpec((1, cdim, cdim), lambda b: (b, 0, 0)),
            pl.BlockSpec((1, cdim, 1), lambda b: (b, 0, 0)),
        ],
        out_shape=[
            jax.ShapeDtypeStruct((bp, cdim, cdim), jnp.float32),
            jax.ShapeDtypeStruct((bp, cdim, 1), jnp.float32),
        ],
        compiler_params=pltpu.CompilerParams(
            dimension_semantics=("arbitrary",)),
    )(src_, corr)

    translation = trans.reshape(bp, cdim)
    return (rot_mat, translation, src_, corr)
